# baseline (speedup 1.0000x reference)
import functools

import jax
import jax.numpy as jnp
from jax import lax
from jax.experimental import pallas as pl
from jax.experimental.pallas import tpu as pltpu
from jax.experimental.pallas import tpu_sc as plsc


SIZE = 32
CELLS = SIZE * SIZE
BS = 1024
E = 200
F = 64
IDX_SPLIT = 2
IDX_CHUNK = E // IDX_SPLIT
ZROWS = 128
ACTIVE_W = 22
BASE_B = BS // ACTIVE_W
EXTRA = BS - BASE_B * ACTIVE_W
MAXB = BASE_B + 1


def _make_sc_call():
    mesh = plsc.VectorSubcoreMesh(core_axis_name="c", subcore_axis_name="s")
    nc = mesh.num_cores

    @functools.partial(
        pl.kernel,
        out_type=jax.ShapeDtypeStruct((BS, CELLS, F), jnp.float32),
        mesh=mesh,
        compiler_params=pltpu.CompilerParams(use_tc_tiling_on_sc=False),
        scratch_types=[
            pltpu.VMEM((4, IDX_SPLIT, IDX_CHUNK), jnp.int32),
            pltpu.VMEM((2, IDX_SPLIT, IDX_CHUNK, F), jnp.float32),
            pltpu.VMEM_SHARED((ACTIVE_W, CELLS, F), jnp.float32),
            pltpu.VMEM((IDX_CHUNK, F), jnp.float32),
            pltpu.SemaphoreType.DMA,
            pltpu.SemaphoreType.DMA,
            pltpu.SemaphoreType.DMA,
            pltpu.SemaphoreType.DMA,
            pltpu.SemaphoreType.DMA,
            pltpu.SemaphoreType.DMA,
        ],
    )
    def sc_scatter(
        x_hbm,
        idx_hbm,
        zeros_hbm,
        out_hbm,
        idx_v,
        x_v,
        acc_sh,
        zeros_v,
        sem_i0,
        sem_i1,
        sem_x0,
        sem_x1,
        sem_o0,
        sem_o1,
    ):
        sid = lax.axis_index("s")
        wid = sid * nc + lax.axis_index("c")
        cnt = jnp.where(
            wid < ACTIVE_W, BASE_B + jnp.where(wid < EXTRA, 1, 0), 0
        )
        start = wid * BASE_B + jnp.minimum(wid, EXTRA)
        asid = jnp.minimum(sid, ACTIVE_W // nc - 1)
        sem_i = (sem_i0, sem_i1)
        sem_x = (sem_x0, sem_x1)
        sem_o = (sem_o0, sem_o1)

        def acc(p):
            return acc_sh.at[2 * asid + p]

        @pl.when(cnt > 0)
        def _():
            pltpu.sync_copy(zeros_hbm.at[pl.ds(0, IDX_CHUNK)], zeros_v)
            for p in range(2):
                for k in range(CELLS // ZROWS):
                    pltpu.sync_copy(zeros_hbm, acc(p).at[pl.ds(k * ZROWS, ZROWS)])

        def start_load(i):
            @pl.when(cnt > i)
            def _():
                b = start + i
                pltpu.async_copy(idx_hbm.at[b], idx_v.at[i % 4], sem_i[i % 2])
                pltpu.async_copy(x_hbm.at[b], x_v.at[i % 2], sem_x[i % 2])

        def wait_load(i):
            b = start + i
            pltpu.make_async_copy(idx_hbm.at[b], idx_v.at[i % 4], sem_i[i % 2]).wait()
            pltpu.make_async_copy(x_hbm.at[b], x_v.at[i % 2], sem_x[i % 2]).wait()

        def wait_out(i):
            pltpu.make_async_copy(
                acc(i % 2), out_hbm.at[start + i], sem_o[i % 2]
            ).wait()

        start_load(0)
        for i in range(MAXB):
            if i + 1 < MAXB:
                start_load(i + 1)

            @pl.when(cnt > i)
            def _(i=i):
                wait_load(i)
                if i >= 2:
                    wait_out(i - 2)
                    for j in range(IDX_SPLIT):
                        pltpu.sync_copy(
                            zeros_v,
                            acc(i % 2).at[idx_v.at[(i - 2) % 4].at[j]],
                        )
                for j in range(IDX_SPLIT):
                    pltpu.sync_copy(
                        x_v.at[i % 2].at[j],
                        acc(i % 2).at[idx_v.at[i % 4].at[j]],
                        add=True,
                    )
                pltpu.async_copy(acc(i % 2), out_hbm.at[start + i], sem_o[i % 2])

        for i in (MAXB - 3, MAXB - 2, MAXB - 1):
            @pl.when(jnp.logical_and(cnt > i, i >= cnt - 2))
            def _(i=i):
                wait_out(i)

    return sc_scatter


def kernel(x, indices):
    idx32 = indices.astype(jnp.int32).reshape(BS, IDX_SPLIT, IDX_CHUNK)
    x4 = x.reshape(BS, IDX_SPLIT, IDX_CHUNK, F)
    zeros = jnp.zeros((ZROWS, F), jnp.float32)
    out = _make_sc_call()(x4, idx32, zeros)
    return out.reshape(BS, SIZE, SIZE, F)


# speedup vs baseline: 1.0031x; 1.0031x over previous
import functools

import jax
import jax.numpy as jnp
from jax import lax
from jax.experimental import pallas as pl
from jax.experimental.pallas import tpu as pltpu
from jax.experimental.pallas import tpu_sc as plsc


SIZE = 32
CELLS = SIZE * SIZE
BS = 1024
E = 200
F = 64
IDX_SPLIT = 2
IDX_CHUNK = E // IDX_SPLIT
ZROWS = 128
ACTIVE_W = 22
BASE_B = BS // ACTIVE_W
EXTRA = BS - BASE_B * ACTIVE_W
MAXB = BASE_B + 1


def _make_sc_call():
    mesh = plsc.VectorSubcoreMesh(core_axis_name="c", subcore_axis_name="s")
    nc = mesh.num_cores

    @functools.partial(
        pl.kernel,
        out_type=jax.ShapeDtypeStruct((BS, SIZE, SIZE, F), jnp.float32),
        mesh=mesh,
        compiler_params=pltpu.CompilerParams(use_tc_tiling_on_sc=False),
        scratch_types=[
            pltpu.VMEM((4, IDX_SPLIT, IDX_CHUNK), jnp.int32),
            pltpu.VMEM((2, E, F), jnp.float32),
            pltpu.VMEM_SHARED((ACTIVE_W, CELLS, F), jnp.float32),
            pltpu.VMEM((IDX_CHUNK, F), jnp.float32),
            pltpu.SemaphoreType.DMA,
            pltpu.SemaphoreType.DMA,
            pltpu.SemaphoreType.DMA,
            pltpu.SemaphoreType.DMA,
            pltpu.SemaphoreType.DMA,
            pltpu.SemaphoreType.DMA,
        ],
    )
    def sc_scatter(
        x_hbm,
        idx_hbm,
        zeros_hbm,
        out_hbm,
        idx_v,
        x_v,
        acc_sh,
        zeros_v,
        sem_i0,
        sem_i1,
        sem_x0,
        sem_x1,
        sem_o0,
        sem_o1,
    ):
        sid = lax.axis_index("s")
        wid = sid * nc + lax.axis_index("c")
        cnt = jnp.where(
            wid < ACTIVE_W, BASE_B + jnp.where(wid < EXTRA, 1, 0), 0
        )
        start = wid * BASE_B + jnp.minimum(wid, EXTRA)
        asid = jnp.minimum(sid, ACTIVE_W // nc - 1)
        sem_i = (sem_i0, sem_i1)
        sem_x = (sem_x0, sem_x1)
        sem_o = (sem_o0, sem_o1)

        def acc(p):
            return acc_sh.at[2 * asid + p]

        @pl.when(cnt > 0)
        def _():
            pltpu.sync_copy(zeros_hbm.at[pl.ds(0, IDX_CHUNK)], zeros_v)
            for p in range(2):
                for k in range(CELLS // ZROWS):
                    pltpu.sync_copy(zeros_hbm, acc(p).at[pl.ds(k * ZROWS, ZROWS)])

        def start_load(i, p):
            @pl.when(i < cnt)
            def _():
                b = start + i
                pltpu.async_copy(idx_hbm.at[b], idx_v.at[i % 4], sem_i[p])
                pltpu.async_copy(x_hbm.at[b], x_v.at[p], sem_x[p])

        def wait_load(p):
            pltpu.make_async_copy(idx_hbm.at[0], idx_v.at[0], sem_i[p]).wait()
            pltpu.make_async_copy(x_hbm.at[0], x_v.at[0], sem_x[p]).wait()

        def start_out(i, p):
            b = start + i
            for r in range(SIZE):
                pltpu.async_copy(
                    acc(p).at[pl.ds(r * SIZE, SIZE)],
                    out_hbm.at[b].at[r],
                    sem_o[p],
                )

        def wait_out(p):
            for r in range(SIZE):
                pltpu.make_async_copy(
                    acc(p).at[pl.ds(r * SIZE, SIZE)],
                    out_hbm.at[0].at[r],
                    sem_o[p],
                ).wait()

        def rezero(i, p):
            slot = i % 4
            for j in range(IDX_SPLIT):
                pltpu.sync_copy(zeros_v, acc(p).at[idx_v.at[slot].at[j]])

        def scatter(i, p):
            slot = i % 4
            for j in range(IDX_SPLIT):
                pltpu.sync_copy(
                    x_v.at[p].at[pl.ds(j * IDX_CHUNK, IDX_CHUNK)],
                    acc(p).at[idx_v.at[slot].at[j]],
                    add=True,
                )

        start_load(jnp.int32(0), 0)
        start_load(jnp.int32(1), 1)

        def pair_body(k, carry):
            i0 = 2 * k
            i1 = 2 * k + 1

            @pl.when(jnp.logical_and(k >= 1, i0 < cnt))
            def _():
                wait_out(0)
                rezero(i0 - 2, 0)

            @pl.when(i0 < cnt)
            def _():
                wait_load(0)
                scatter(i0, 0)
                start_out(i0, 0)

            start_load(i0 + 2, 0)

            @pl.when(jnp.logical_and(k >= 1, i1 < cnt))
            def _():
                wait_out(1)
                rezero(i1 - 2, 1)

            @pl.when(i1 < cnt)
            def _():
                wait_load(1)
                scatter(i1, 1)
                start_out(i1, 1)

            start_load(i1 + 2, 1)

            return carry

        lax.fori_loop(0, (cnt + 1) // 2, pair_body, 0)

        @pl.when(cnt >= 1)
        def _():
            wait_out(0)

        @pl.when(cnt >= 2)
        def _():
            wait_out(1)

    return sc_scatter


def kernel(x, indices):
    idx32 = indices.astype(jnp.int32).reshape(BS, IDX_SPLIT, IDX_CHUNK)
    zeros = jnp.zeros((ZROWS, F), jnp.float32)
    return _make_sc_call()(x, idx32, zeros)
